# Initial kernel scaffold; baseline (speedup 1.0000x reference)
import functools

import jax
import jax.numpy as jnp
from jax import lax
from jax.experimental import pallas as pl
from jax.experimental.pallas import tpu as pltpu
from jax.experimental.pallas import tpu_sc as plsc

N = 10000
NP = 10240
E = 160000
F = 256
HALF = 128
G = 128
NSUB = 16
ROWS_PT = NP // NSUB
ECH = 80
ECHUNKS = E // NSUB // ECH
POOL_ROWS = 136


def _edge_mesh_kernel():
  mesh = plsc.VectorSubcoreMesh(core_axis_name="c", subcore_axis_name="s")

  @functools.partial(
      pl.kernel,
      mesh=mesh,
      out_type=jax.ShapeDtypeStruct((2, NP, HALF), jnp.float32),
      scratch_types=[
          pltpu.VMEM((ECHUNKS, ECH), jnp.int32),
          pltpu.VMEM((ECHUNKS, ECH), jnp.int32),
          pltpu.VMEM((ECH, HALF), jnp.float32),
          pltpu.VMEM_SHARED((NP, HALF), jnp.float32),
          pltpu.SemaphoreType.DMA,
      ],
  )
  def edge_kernel(hlo, hhi, src2, dst2, out, src_v, dst_v, rows_v, agg_sh,
                  sem):
    cid = lax.axis_index("c")
    sid = lax.axis_index("s")
    r0 = sid * ROWS_PT

    def init_from(h_ref):
      for j in range(ROWS_PT // 128):
        sl = pl.ds(r0 + j * 128, 128)
        pltpu.sync_copy(h_ref.at[sl], agg_sh.at[sl])

    @pl.when(cid == 0)
    def _():
      init_from(hlo)

    @pl.when(cid == 1)
    def _():
      init_from(hhi)

    pltpu.sync_copy(src2.at[pl.ds(sid * ECHUNKS, ECHUNKS)], src_v)
    pltpu.sync_copy(dst2.at[pl.ds(sid * ECHUNKS, ECHUNKS)], dst_v)
    plsc.subcore_barrier()

    def edges_from(h_ref):
      def body(c, carry):
        pltpu.async_copy(h_ref.at[src_v.at[c]], rows_v, sem).wait()
        pltpu.sync_copy(rows_v, agg_sh.at[dst_v.at[c]], add=True)
        return carry

      lax.fori_loop(0, ECHUNKS, body, 0)

    @pl.when(cid == 0)
    def _():
      edges_from(hlo)

    @pl.when(cid == 1)
    def _():
      edges_from(hhi)

    plsc.subcore_barrier()
    for j in range(ROWS_PT // 128):
      sl = pl.ds(r0 + j * 128, 128)
      pltpu.sync_copy(agg_sh.at[sl], out.at[cid].at[sl])

  return edge_kernel


def _pool_mesh_kernel():
  mesh = plsc.VectorSubcoreMesh(core_axis_name="c", subcore_axis_name="s")

  @functools.partial(
      pl.kernel,
      mesh=mesh,
      out_type=(
          jax.ShapeDtypeStruct((2, G, HALF), jnp.float32),
          jax.ShapeDtypeStruct((G, HALF), jnp.float32),
      ),
      scratch_types=[
          pltpu.VMEM((NP // NSUB // 128, 128), jnp.int32),
          pltpu.VMEM((128, HALF), jnp.float32),
          pltpu.VMEM((128, HALF), jnp.float32),
          pltpu.VMEM_SHARED((POOL_ROWS, HALF), jnp.float32),
          pltpu.VMEM_SHARED((POOL_ROWS, HALF), jnp.float32),
      ],
  )
  def pool_kernel(hlo, hhi, bat2, zeros, ones, sums_out, cnt_out, bat_v,
                  rows_v, ones_v, pool_sh, cnt_sh):
    cid = lax.axis_index("c")
    sid = lax.axis_index("s")
    nchunks = ROWS_PT // 128

    @pl.when(sid == 0)
    def _():
      pltpu.sync_copy(zeros, pool_sh)
      pltpu.sync_copy(zeros, cnt_sh)

    pltpu.sync_copy(bat2.at[pl.ds(sid * nchunks, nchunks)], bat_v)
    pltpu.sync_copy(ones, ones_v)
    plsc.subcore_barrier()

    def pool_from(h_ref):
      def body(j, carry):
        pltpu.sync_copy(h_ref.at[pl.ds(sid * ROWS_PT + j * 128, 128)], rows_v)
        pltpu.sync_copy(rows_v, pool_sh.at[bat_v.at[j]], add=True)
        pltpu.sync_copy(ones_v, cnt_sh.at[bat_v.at[j]], add=True)
        return carry

      lax.fori_loop(0, nchunks, body, 0)

    @pl.when(cid == 0)
    def _():
      pool_from(hlo)

    @pl.when(cid == 1)
    def _():
      pool_from(hhi)

    plsc.subcore_barrier()
    sl = pl.ds(sid * 8, 8)
    pltpu.sync_copy(pool_sh.at[sl], sums_out.at[cid].at[sl])

    @pl.when(cid == 0)
    def _():
      pltpu.sync_copy(cnt_sh.at[sl], cnt_out.at[sl])

  return pool_kernel


_EDGE_KERNEL = _edge_mesh_kernel()
_POOL_KERNEL = _pool_mesh_kernel()


def _mlp_body(relu_out, lo_ref, hi_ref, w1a_ref, w1b_ref, b1_ref, w2_ref,
              b2_ref, olo_ref, ohi_ref):
  pre = (
      jnp.dot(lo_ref[...], w1a_ref[...], preferred_element_type=jnp.float32)
      + jnp.dot(hi_ref[...], w1b_ref[...], preferred_element_type=jnp.float32)
      + b1_ref[...]
  )
  act = jnp.maximum(pre, 0.0)
  out = jnp.dot(act, w2_ref[...], preferred_element_type=jnp.float32) + b2_ref[...]
  if relu_out:
    out = jnp.maximum(out, 0.0)
  olo_ref[...] = out[:, :HALF]
  ohi_ref[...] = out[:, HALF:]


def _mlp(lo, hi, W1, b1, W2, b2, relu_out):
  BN = 640
  grid = NP // BN
  row_spec = pl.BlockSpec((BN, HALF), lambda i: (i, 0))
  return pl.pallas_call(
      functools.partial(_mlp_body, relu_out),
      grid=(grid,),
      in_specs=[
          row_spec,
          row_spec,
          pl.BlockSpec((HALF, F), lambda i: (0, 0)),
          pl.BlockSpec((HALF, F), lambda i: (0, 0)),
          pl.BlockSpec((1, F), lambda i: (0, 0)),
          pl.BlockSpec((F, F), lambda i: (0, 0)),
          pl.BlockSpec((1, F), lambda i: (0, 0)),
      ],
      out_specs=[row_spec, row_spec],
      out_shape=[
          jax.ShapeDtypeStruct((NP, HALF), jnp.float32),
          jax.ShapeDtypeStruct((NP, HALF), jnp.float32),
      ],
  )(lo, hi, W1[:HALF], W1[HALF:], b1.reshape(1, F), W2, b2.reshape(1, F))


def _div_body(s0_ref, s1_ref, c_ref, out_ref):
  cnt = jnp.maximum(c_ref[:, 0:1], 1.0)
  out_ref[:, :HALF] = s0_ref[...] / cnt
  out_ref[:, HALF:] = s1_ref[...] / cnt


def _div(sums, counts):
  return pl.pallas_call(
      _div_body,
      out_shape=jax.ShapeDtypeStruct((G, F), jnp.float32),
  )(sums[0], sums[1], counts)


@jax.jit
def kernel(x, edge_index, batch, W1_0, b1_0, W2_0, b2_0, W1_1, b1_1, W2_1,
           b2_1, W1_2, b1_2, W2_2, b2_2):
  xp = jnp.pad(x, ((0, NP - N), (0, 0)))
  lo, hi = xp[:, :HALF], xp[:, HALF:]
  src2 = edge_index[0].reshape(E // ECH, ECH)
  dst2 = edge_index[1].reshape(E // ECH, ECH)
  bat2 = jnp.pad(batch, (0, NP - N), constant_values=G).reshape(NP // 128, 128)
  zeros = jnp.zeros((POOL_ROWS, HALF), jnp.float32)
  ones = jnp.ones((128, HALF), jnp.float32)

  params = [(W1_0, b1_0, W2_0, b2_0), (W1_1, b1_1, W2_1, b2_1),
            (W1_2, b1_2, W2_2, b2_2)]
  for i, (W1, b1, W2, b2) in enumerate(params):
    agg = _EDGE_KERNEL(lo, hi, src2, dst2)
    lo, hi = _mlp(agg[0], agg[1], W1, b1, W2, b2, relu_out=(i < 2))

  sums, counts = _POOL_KERNEL(lo, hi, bat2, zeros, ones)
  return _div(sums, counts)


# speedup vs baseline: 4.4238x; 4.4238x over previous
import functools

import jax
import jax.numpy as jnp
from jax import lax
from jax.experimental import pallas as pl
from jax.experimental.pallas import tpu as pltpu
from jax.experimental.pallas import tpu_sc as plsc

N = 10000
NP = 10240
E = 160000
F = 256
HALF = 128
G = 128
NSUB = 16
ROWS_PT = NP // NSUB
ECH = 80
ECHUNKS = E // NSUB // ECH
POOL_ROWS = 136


def _edge_mesh_kernel():
  mesh = plsc.VectorSubcoreMesh(core_axis_name="c", subcore_axis_name="s")

  @functools.partial(
      pl.kernel,
      mesh=mesh,
      out_type=jax.ShapeDtypeStruct((2, NP, HALF), jnp.float32),
      scratch_types=[
          pltpu.VMEM((ECHUNKS, ECH), jnp.int32),
          pltpu.VMEM((ECHUNKS, ECH), jnp.int32),
          pltpu.VMEM((ECH, HALF), jnp.float32),
          pltpu.VMEM_SHARED((NP, HALF), jnp.float32),
          pltpu.SemaphoreType.DMA,
      ],
  )
  def edge_kernel(hlo, hhi, src2, dst2, out, src_v, dst_v, rows_v, agg_sh,
                  sem):
    cid = lax.axis_index("c")
    sid = lax.axis_index("s")
    r0 = sid * ROWS_PT

    def init_from(h_ref):
      for j in range(ROWS_PT // 128):
        sl = pl.ds(r0 + j * 128, 128)
        pltpu.sync_copy(h_ref.at[sl], agg_sh.at[sl])

    @pl.when(cid == 0)
    def _():
      init_from(hlo)

    @pl.when(cid == 1)
    def _():
      init_from(hhi)

    pltpu.sync_copy(src2.at[sid], src_v)
    pltpu.sync_copy(dst2.at[sid], dst_v)
    plsc.subcore_barrier()

    def edges_from(h_ref):
      def body(c, carry):
        pltpu.async_copy(h_ref.at[src_v.at[c]], rows_v, sem).wait()
        pltpu.sync_copy(rows_v, agg_sh.at[dst_v.at[c]], add=True)
        return carry

      lax.fori_loop(0, ECHUNKS, body, 0)

    @pl.when(cid == 0)
    def _():
      edges_from(hlo)

    @pl.when(cid == 1)
    def _():
      edges_from(hhi)

    plsc.subcore_barrier()
    for j in range(ROWS_PT // 128):
      sl = pl.ds(r0 + j * 128, 128)
      pltpu.sync_copy(agg_sh.at[sl], out.at[cid].at[sl])

  return edge_kernel


def _pool_mesh_kernel():
  mesh = plsc.VectorSubcoreMesh(core_axis_name="c", subcore_axis_name="s")

  @functools.partial(
      pl.kernel,
      mesh=mesh,
      out_type=(
          jax.ShapeDtypeStruct((2, G, HALF), jnp.float32),
          jax.ShapeDtypeStruct((G, HALF), jnp.float32),
      ),
      scratch_types=[
          pltpu.VMEM((NP // NSUB // 128, 128), jnp.int32),
          pltpu.VMEM((128, HALF), jnp.float32),
          pltpu.VMEM((128, HALF), jnp.float32),
          pltpu.VMEM_SHARED((POOL_ROWS, HALF), jnp.float32),
          pltpu.VMEM_SHARED((POOL_ROWS, HALF), jnp.float32),
      ],
  )
  def pool_kernel(hlo, hhi, bat2, zeros, ones, sums_out, cnt_out, bat_v,
                  rows_v, ones_v, pool_sh, cnt_sh):
    cid = lax.axis_index("c")
    sid = lax.axis_index("s")
    nchunks = ROWS_PT // 128

    @pl.when(sid == 0)
    def _():
      pltpu.sync_copy(zeros, pool_sh)
      pltpu.sync_copy(zeros, cnt_sh)

    pltpu.sync_copy(bat2.at[sid], bat_v)
    pltpu.sync_copy(ones, ones_v)
    plsc.subcore_barrier()

    def pool_from(h_ref):
      def body(j, carry):
        pltpu.sync_copy(h_ref.at[pl.ds(sid * ROWS_PT + j * 128, 128)], rows_v)
        pltpu.sync_copy(rows_v, pool_sh.at[bat_v.at[j]], add=True)
        pltpu.sync_copy(ones_v, cnt_sh.at[bat_v.at[j]], add=True)
        return carry

      lax.fori_loop(0, nchunks, body, 0)

    @pl.when(cid == 0)
    def _():
      pool_from(hlo)

    @pl.when(cid == 1)
    def _():
      pool_from(hhi)

    plsc.subcore_barrier()
    sl = pl.ds(sid * 8, 8)
    pltpu.sync_copy(pool_sh.at[sl], sums_out.at[cid].at[sl])

    @pl.when(cid == 0)
    def _():
      pltpu.sync_copy(cnt_sh.at[sl], cnt_out.at[sl])

  return pool_kernel


_EDGE_KERNEL = _edge_mesh_kernel()
_POOL_KERNEL = _pool_mesh_kernel()


def _mlp_body(relu_out, lo_ref, hi_ref, w1a_ref, w1b_ref, b1_ref, w2_ref,
              b2_ref, olo_ref, ohi_ref):
  pre = (
      jnp.dot(lo_ref[...], w1a_ref[...], preferred_element_type=jnp.float32)
      + jnp.dot(hi_ref[...], w1b_ref[...], preferred_element_type=jnp.float32)
      + b1_ref[...]
  )
  act = jnp.maximum(pre, 0.0)
  out = jnp.dot(act, w2_ref[...], preferred_element_type=jnp.float32) + b2_ref[...]
  if relu_out:
    out = jnp.maximum(out, 0.0)
  olo_ref[...] = out[:, :HALF]
  ohi_ref[...] = out[:, HALF:]


def _mlp(lo, hi, W1, b1, W2, b2, relu_out):
  BN = 640
  grid = NP // BN
  row_spec = pl.BlockSpec((BN, HALF), lambda i: (i, 0))
  return pl.pallas_call(
      functools.partial(_mlp_body, relu_out),
      grid=(grid,),
      in_specs=[
          row_spec,
          row_spec,
          pl.BlockSpec((HALF, F), lambda i: (0, 0)),
          pl.BlockSpec((HALF, F), lambda i: (0, 0)),
          pl.BlockSpec((1, F), lambda i: (0, 0)),
          pl.BlockSpec((F, F), lambda i: (0, 0)),
          pl.BlockSpec((1, F), lambda i: (0, 0)),
      ],
      out_specs=[row_spec, row_spec],
      out_shape=[
          jax.ShapeDtypeStruct((NP, HALF), jnp.float32),
          jax.ShapeDtypeStruct((NP, HALF), jnp.float32),
      ],
  )(lo, hi, W1[:HALF], W1[HALF:], b1.reshape(1, F), W2, b2.reshape(1, F))


def _div_body(s0_ref, s1_ref, c_ref, out_ref):
  cnt = jnp.maximum(c_ref[:, 0:1], 1.0)
  out_ref[:, :HALF] = s0_ref[...] / cnt
  out_ref[:, HALF:] = s1_ref[...] / cnt


def _div(sums, counts):
  return pl.pallas_call(
      _div_body,
      out_shape=jax.ShapeDtypeStruct((G, F), jnp.float32),
  )(sums[0], sums[1], counts)


@jax.jit
def kernel(x, edge_index, batch, W1_0, b1_0, W2_0, b2_0, W1_1, b1_1, W2_1,
           b2_1, W1_2, b1_2, W2_2, b2_2):
  xp = jnp.pad(x, ((0, NP - N), (0, 0)))
  lo, hi = xp[:, :HALF], xp[:, HALF:]
  src2 = edge_index[0].reshape(NSUB, ECHUNKS, ECH)
  dst2 = edge_index[1].reshape(NSUB, ECHUNKS, ECH)
  bat2 = jnp.pad(batch, (0, NP - N), constant_values=G).reshape(
      NSUB, ROWS_PT // 128, 128)
  zeros = jnp.zeros((POOL_ROWS, HALF), jnp.float32)
  ones = jnp.ones((128, HALF), jnp.float32)

  params = [(W1_0, b1_0, W2_0, b2_0), (W1_1, b1_1, W2_1, b2_1),
            (W1_2, b1_2, W2_2, b2_2)]
  for i, (W1, b1, W2, b2) in enumerate(params):
    agg = _EDGE_KERNEL(lo, hi, src2, dst2)
    lo, hi = _mlp(agg[0], agg[1], W1, b1, W2, b2, relu_out=(i < 2))

  sums, counts = _POOL_KERNEL(lo, hi, bat2, zeros, ones)
  return _div(sums, counts)
